# baseline (speedup 1.0000x reference)
import functools

import jax
import jax.numpy as jnp
from jax import lax
from jax.experimental import pallas as pl
from jax.experimental.pallas import tpu as pltpu
from jax.experimental.pallas import tpu_sc as plsc

NU = 25000
NI = 25000
NN = NU + NI
NNP = 51200
NE = 800000
NEP = 819200
D = 64
H = 32
B = 4096
NL = 3
DECAY = 1e-4

NBLK = 64
NUB = 391
EPT = NEP // 16
SB = 128
NSB = EPT // SB
RPT = NNP // 16
SPT = B // 16

_f32 = jnp.float32
_i32 = jnp.int32


def _sc_body(users_h, pos_h, neg_h, epk_h,
             ug_h, ua_h, ic_h, ue_h, ie_h, ge_h, ae_h, ce_h,
             xs_h, mean_h, reg_h,
             acc, gtab, atab, ctab, ublock, outblock,
             rows0, rows1, rows2, rows3, eb0, eb1, eb2, eb3, idxb,
             esem0, esem1, esem2, esem3, gsem0, gsem1, gsem2, gsem3,
             xsem0, xsem1, xsem2, xsem3, usem):
  cid = lax.axis_index("c")
  sid = lax.axis_index("s")
  cb = cid * H
  rows_ = (rows0, rows1, rows2, rows3)
  eb_ = (eb0, eb1, eb2, eb3)
  esem_ = (esem0, esem1, esem2, esem3)
  gsem_ = (gsem0, gsem1, gsem2, gsem3)
  xsem_ = (xsem0, xsem1, xsem2, xsem3)

  pltpu.sync_copy(ge_h, gtab)
  pltpu.sync_copy(ae_h, atab)
  pltpu.sync_copy(ce_h, ctab)

  gidx_b = idxb.at[pl.ds(0, NBLK)]
  aidx_b = idxb.at[pl.ds(NBLK, NBLK)]

  def _build_user(b):
    n0 = jnp.minimum(NBLK * b, NU - NBLK)
    d1 = pltpu.async_copy(ue_h.at[pl.ds(n0, NBLK)], ublock, usem)
    d2 = pltpu.async_copy(ug_h.at[pl.ds(n0, NBLK)], gidx_b, usem)
    d3 = pltpu.async_copy(ua_h.at[pl.ds(n0, NBLK)], aidx_b, usem)
    d1.wait(); d2.wait(); d3.wait()

    def _grp(g, _):
      gv = gidx_b[pl.ds(16 * g, 16)]
      av = aidx_b[pl.ds(16 * g, 16)]
      for j in range(16):
        n = 16 * g + j
        gj = gv[j]
        aj = av[j]
        for h2 in range(2):
          sl = pl.ds(16 * h2, 16)
          outblock[n, sl] = (ublock[n, pl.ds(cb + 16 * h2, 16)]
                             + gtab[gj, pl.ds(cb + 16 * h2, 16)]
                             + atab[aj, pl.ds(cb + 16 * h2, 16)])
      return _
    lax.fori_loop(0, NBLK // 16, _grp, None)
    pltpu.sync_copy(outblock, xs_h.at[0, cid, pl.ds(n0, NBLK)])

  def _build_item(b):
    n0 = jnp.minimum(NBLK * b, NI - NBLK)
    d1 = pltpu.async_copy(ie_h.at[pl.ds(n0, NBLK)], ublock, usem)
    d2 = pltpu.async_copy(ic_h.at[pl.ds(n0, NBLK)], gidx_b, usem)
    d1.wait(); d2.wait()

    def _grp(g, _):
      cv = gidx_b[pl.ds(16 * g, 16)]
      for j in range(16):
        n = 16 * g + j
        cj = cv[j]
        for h2 in range(2):
          sl = pl.ds(16 * h2, 16)
          outblock[n, sl] = (ublock[n, pl.ds(cb + 16 * h2, 16)]
                             + ctab[cj, pl.ds(cb + 16 * h2, 16)])
      return _
    lax.fori_loop(0, NBLK // 16, _grp, None)
    pltpu.sync_copy(outblock, xs_h.at[0, cid, pl.ds(NU + n0, NBLK)])

  def _build_k(k, _):
    b = sid + 16 * k

    @pl.when(b < NUB)
    def _():
      _build_user(b)
      _build_item(b)
    return _
  lax.fori_loop(0, 25, _build_k, None)

  plsc.subcore_barrier()

  def _zb(r, _):
    outblock[r, pl.ds(0, 16)] = jnp.zeros((16,), _f32)
    outblock[r, pl.ds(16, 16)] = jnp.zeros((16,), _f32)
    return _
  lax.fori_loop(0, NBLK, _zb, None)

  def _fire_edges(s, e):
    pltpu.async_copy(epk_h.at[sid * NSB + s], eb_[e], esem_[e])

  def _wait_edges(s, e):
    pltpu.make_async_copy(epk_h.at[sid * NSB + s], eb_[e], esem_[e]).wait()

  def _fire_gather(l, e, r):
    pltpu.async_copy(xs_h.at[l, cid].at[eb_[e].at[0]], rows_[r], gsem_[r])

  def _wait_gather(l, e, r):
    pltpu.make_async_copy(xs_h.at[l, cid].at[eb_[e].at[0]],
                          rows_[r], gsem_[r]).wait()

  def _fire_scat(e, r):
    pltpu.async_copy(rows_[r], acc.at[eb_[e].at[1]], xsem_[r], add=True)

  def _wait_scat(e, r):
    pltpu.make_async_copy(rows_[r], acc.at[eb_[e].at[1]], xsem_[r]).wait()

  for l in range(NL):
    zdescs = []
    for z in range(RPT // NBLK):
      zdescs.append(pltpu.async_copy(
          outblock, acc.at[pl.ds(sid * RPT + NBLK * z, NBLK)], usem))
    for dsc in zdescs:
      dsc.wait()
    plsc.subcore_barrier()

    _fire_edges(0, 0)
    _fire_edges(1, 1)
    _wait_edges(0, 0)
    _fire_gather(l, 0, 0)
    _wait_edges(1, 1)
    _fire_gather(l, 1, 1)

    def _quad(i, _):
      for u in range(4):
        s = 4 * i + u
        e = u
        e2 = (u + 2) % 4

        @pl.when(s >= 2)
        def _():
          _wait_scat(e2, e2)

        @pl.when(s + 2 < NSB)
        def _():
          _fire_edges(s + 2, e2)

        _wait_gather(l, e, u)

        def _grp(eg, __):
          w16 = plsc.bitcast(eb_[e][2, pl.ds(16 * eg, 16)], _f32)
          base = 16 * eg
          for j in range(16):
            wj = jnp.broadcast_to(w16[j], (16,))
            for h2 in range(2):
              sl = pl.ds(16 * h2, 16)
              rows_[u][base + j, sl] = rows_[u][base + j, sl] * wj
          return __
        lax.fori_loop(0, SB // 16, _grp, None)

        _fire_scat(e, u)

        @pl.when(s + 2 < NSB)
        def _():
          _wait_edges(s + 2, e2)
          _fire_gather(l, e2, e2)
      return _
    lax.fori_loop(0, NSB // 4, _quad, None)
    _wait_scat(2, 2)
    _wait_scat(3, 3)
    plsc.subcore_barrier()

    cdescs = []
    for z in range(12):
      rr = sid * RPT + 256 * z
      cdescs.append(pltpu.async_copy(
          acc.at[pl.ds(rr, 256)], xs_h.at[l + 1, cid, pl.ds(rr, 256)], usem))
    cdescs.append(pltpu.async_copy(
        acc.at[pl.ds(sid * RPT + 3072, 128)],
        xs_h.at[l + 1, cid, pl.ds(sid * RPT + 3072, 128)], usem))
    for dsc in cdescs:
      dsc.wait()
    plsc.subcore_barrier()

  for ridx, idx_h in enumerate((users_h, pos_h, neg_h)):
    for t in range(SPT // 128):
      s0 = sid * SPT + 128 * t
      pltpu.sync_copy(idx_h.at[pl.ds(s0, 128)], idxb)
      if ridx > 0:
        def _off(i, _):
          v = idxb[pl.ds(16 * i, 16)]
          idxb[pl.ds(16 * i, 16)] = v + NU
          return _
        lax.fori_loop(0, 8, _off, None)
      descs = []
      for l4 in range(NL + 1):
        descs.append(pltpu.async_copy(
            xs_h.at[l4, cid].at[idxb], rows_[l4], gsem_[l4]))
      for dsc in descs:
        dsc.wait()

      for c in range(2):
        def _mrow(r, _):
          rr = 64 * c + r
          for h2 in range(2):
            sl = pl.ds(16 * h2, 16)
            v = (rows0[rr, sl] + rows1[rr, sl]
                 + rows2[rr, sl] + rows3[rr, sl]) * 0.25
            outblock[r, sl] = v
          return _
        lax.fori_loop(0, 64, _mrow, None)
        pltpu.sync_copy(outblock,
                        mean_h.at[ridx, cid, pl.ds(s0 + 64 * c, 64)])

  for ridx, (idx_h, tbl_h) in enumerate(((users_h, ue_h),
                                         (pos_h, ie_h),
                                         (neg_h, ie_h))):
    s0 = cid * (B // 2) + sid * 128
    pltpu.sync_copy(idx_h.at[pl.ds(s0, 128)], idxb)
    for c in range(2):
      pltpu.async_copy(tbl_h.at[idxb.at[pl.ds(64 * c, 64)]],
                       ublock, usem).wait()
      pltpu.sync_copy(ublock, reg_h.at[ridx, pl.ds(s0 + 64 * c, 64)])


_sc_forward = pl.kernel(
    _sc_body,
    out_type=(
        jax.ShapeDtypeStruct((NL + 1, 2, NNP, H), _f32),
        jax.ShapeDtypeStruct((3, 2, B, H), _f32),
        jax.ShapeDtypeStruct((3, B, D), _f32),
    ),
    mesh=plsc.VectorSubcoreMesh(core_axis_name="c", subcore_axis_name="s",
                                num_cores=2, num_subcores=16),
    compiler_params=pltpu.CompilerParams(needs_layout_passes=False,
                                         use_tc_tiling_on_sc=False),
    scratch_types=[
        pltpu.VMEM_SHARED((NNP, H), _f32),
        pltpu.VMEM((3, D), _f32),
        pltpu.VMEM((8, D), _f32),
        pltpu.VMEM((11, D), _f32),
        pltpu.VMEM((NBLK, D), _f32),
        pltpu.VMEM((NBLK, H), _f32),
        pltpu.VMEM((SB, H), _f32),
        pltpu.VMEM((SB, H), _f32),
        pltpu.VMEM((SB, H), _f32),
        pltpu.VMEM((SB, H), _f32),
        pltpu.VMEM((3, 128), _i32),
        pltpu.VMEM((3, 128), _i32),
        pltpu.VMEM((3, 128), _i32),
        pltpu.VMEM((3, 128), _i32),
        pltpu.VMEM((128,), _i32),
        pltpu.SemaphoreType.DMA,
        pltpu.SemaphoreType.DMA,
        pltpu.SemaphoreType.DMA,
        pltpu.SemaphoreType.DMA,
        pltpu.SemaphoreType.DMA,
        pltpu.SemaphoreType.DMA,
        pltpu.SemaphoreType.DMA,
        pltpu.SemaphoreType.DMA,
        pltpu.SemaphoreType.DMA,
        pltpu.SemaphoreType.DMA,
        pltpu.SemaphoreType.DMA,
        pltpu.SemaphoreType.DMA,
        pltpu.SemaphoreType.DMA,
    ],
)


def _loss_body(mean_ref, reg_ref, out_ref):
  u = mean_ref[0]
  pi = mean_ref[1]
  ni = mean_ref[2]
  ps = jnp.sum(u * pi, axis=(0, 2))
  ns = jnp.sum(u * ni, axis=(0, 2))
  x = ps - ns
  bpr = -jnp.mean(jnp.minimum(x, 0.0) - jnp.log1p(jnp.exp(-jnp.abs(x))))
  r = reg_ref[...]
  reg = jnp.sum(r * r) / B
  out_ref[...] = jnp.reshape(bpr + DECAY * reg, (1, 1))


_tc_loss = pl.pallas_call(
    _loss_body,
    out_shape=jax.ShapeDtypeStruct((1, 1), _f32),
)


@jax.jit
def kernel(users, pos_items, neg_items, edge_index, edge_weight,
           user_gender, user_age_bucket, item_cat,
           user_emb, item_emb, gender_emb, age_emb, cat_emb):
  dst = edge_index[0].astype(_i32)
  src = edge_index[1].astype(_i32)
  pad = NEP - NE
  pad_idx = jnp.arange(pad, dtype=_i32)
  src1 = jnp.concatenate([src, pad_idx % NN])
  dst1 = jnp.concatenate([dst, NN + pad_idx % (NNP - NN)])
  w1 = jnp.concatenate([edge_weight.astype(_f32), jnp.zeros((pad,), _f32)])
  epk = jnp.stack([src1.reshape(-1, 128), dst1.reshape(-1, 128),
                   lax.bitcast_convert_type(w1, _i32).reshape(-1, 128)],
                  axis=1)

  _, mean_rows, reg_rows = _sc_forward(
      users.astype(_i32), pos_items.astype(_i32), neg_items.astype(_i32),
      epk,
      user_gender.astype(_i32), user_age_bucket.astype(_i32),
      item_cat.astype(_i32),
      user_emb.astype(_f32), item_emb.astype(_f32),
      gender_emb.astype(_f32), age_emb.astype(_f32), cat_emb.astype(_f32))

  loss = _tc_loss(mean_rows, reg_rows)
  return jnp.reshape(loss, ())


# speedup vs baseline: 1.1876x; 1.1876x over previous
import functools

import jax
import jax.numpy as jnp
from jax import lax
from jax.experimental import pallas as pl
from jax.experimental.pallas import tpu as pltpu
from jax.experimental.pallas import tpu_sc as plsc

NU = 25000
NI = 25000
NN = NU + NI
NNP = 51200
NE = 800000
NEP = 819200
D = 64
H = 32
B = 4096
NL = 3
DECAY = 1e-4

NBLK = 64
NUB = 391
EPT = NEP // 16
SB = 256
NSB = EPT // SB
RPT = NNP // 16
SPT = B // 16

_f32 = jnp.float32
_i32 = jnp.int32


def _sc_body(users_h, pos_h, neg_h, epk_h,
             ug_h, ua_h, ic_h, ue_h, ie_h, ge_h, ae_h, ce_h,
             xs_h, mean_h, reg_h,
             acc, gtab, atab, ctab, ublock, outblock,
             rows0, rows1, eb0, eb1, eb2, eb3, idxb,
             esem0, esem1, esem2, esem3, gsem0, gsem1, usem):
  cid = lax.axis_index("c")
  sid = lax.axis_index("s")
  cb = cid * H
  rows_ = (rows0, rows1)
  eb_ = (eb0, eb1, eb2, eb3)
  esem_ = (esem0, esem1, esem2, esem3)
  gsem_ = (gsem0, gsem1)

  pltpu.sync_copy(ge_h, gtab)
  pltpu.sync_copy(ae_h, atab)
  pltpu.sync_copy(ce_h, ctab)

  gidx_b = idxb.at[pl.ds(0, NBLK)]
  aidx_b = idxb.at[pl.ds(NBLK, NBLK)]

  def _build_user(b):
    n0 = jnp.minimum(NBLK * b, NU - NBLK)
    d1 = pltpu.async_copy(ue_h.at[pl.ds(n0, NBLK)], ublock, usem)
    d2 = pltpu.async_copy(ug_h.at[pl.ds(n0, NBLK)], gidx_b, usem)
    d3 = pltpu.async_copy(ua_h.at[pl.ds(n0, NBLK)], aidx_b, usem)
    d1.wait(); d2.wait(); d3.wait()

    def _grp(g, _):
      gv = gidx_b[pl.ds(16 * g, 16)]
      av = aidx_b[pl.ds(16 * g, 16)]
      for j in range(16):
        n = 16 * g + j
        gj = gv[j]
        aj = av[j]
        for h2 in range(2):
          sl = pl.ds(16 * h2, 16)
          outblock[n, sl] = (ublock[n, pl.ds(cb + 16 * h2, 16)]
                             + gtab[gj, pl.ds(cb + 16 * h2, 16)]
                             + atab[aj, pl.ds(cb + 16 * h2, 16)])
      return _
    lax.fori_loop(0, NBLK // 16, _grp, None)
    pltpu.sync_copy(outblock, xs_h.at[0, cid, pl.ds(n0, NBLK)])

  def _build_item(b):
    n0 = jnp.minimum(NBLK * b, NI - NBLK)
    d1 = pltpu.async_copy(ie_h.at[pl.ds(n0, NBLK)], ublock, usem)
    d2 = pltpu.async_copy(ic_h.at[pl.ds(n0, NBLK)], gidx_b, usem)
    d1.wait(); d2.wait()

    def _grp(g, _):
      cv = gidx_b[pl.ds(16 * g, 16)]
      for j in range(16):
        n = 16 * g + j
        cj = cv[j]
        for h2 in range(2):
          sl = pl.ds(16 * h2, 16)
          outblock[n, sl] = (ublock[n, pl.ds(cb + 16 * h2, 16)]
                             + ctab[cj, pl.ds(cb + 16 * h2, 16)])
      return _
    lax.fori_loop(0, NBLK // 16, _grp, None)
    pltpu.sync_copy(outblock, xs_h.at[0, cid, pl.ds(NU + n0, NBLK)])

  def _build_k(k, _):
    b = sid + 16 * k

    @pl.when(b < NUB)
    def _():
      _build_user(b)
      _build_item(b)
    return _
  lax.fori_loop(0, 25, _build_k, None)

  plsc.subcore_barrier()

  def _zb(r, _):
    outblock[r, pl.ds(0, 16)] = jnp.zeros((16,), _f32)
    outblock[r, pl.ds(16, 16)] = jnp.zeros((16,), _f32)
    return _
  lax.fori_loop(0, NBLK, _zb, None)

  def _fire_edges(s, e):
    pltpu.async_copy(epk_h.at[sid * NSB + s], eb_[e], esem_[e])

  def _wait_edges(s, e):
    pltpu.make_async_copy(epk_h.at[sid * NSB + s], eb_[e], esem_[e]).wait()

  def _fire_gathers(l, e, p):
    for j in range(2):
      pltpu.async_copy(xs_h.at[l, cid].at[eb_[e].at[j]],
                       rows_[p].at[pl.ds(128 * j, 128)], gsem_[p])

  def _wait_gathers(l, e, p):
    for j in range(2):
      pltpu.make_async_copy(xs_h.at[l, cid].at[eb_[e].at[j]],
                            rows_[p].at[pl.ds(128 * j, 128)], gsem_[p]).wait()

  for l in range(NL):
    zdescs = []
    for z in range(RPT // NBLK):
      zdescs.append(pltpu.async_copy(
          outblock, acc.at[pl.ds(sid * RPT + NBLK * z, NBLK)], usem))
    for dsc in zdescs:
      dsc.wait()
    plsc.subcore_barrier()

    _fire_edges(0, 0)
    _fire_edges(1, 1)
    _wait_edges(0, 0)
    _fire_gathers(l, 0, 0)
    _wait_edges(1, 1)
    _fire_gathers(l, 1, 1)

    def _quad(i, _):
      for u in range(4):
        s = 4 * i + u
        p = u % 2
        e = u
        e2 = (u + 2) % 4

        @pl.when(s + 2 < NSB)
        def _():
          _fire_edges(s + 2, e2)

        _wait_gathers(l, e, p)

        def _grp(eg, __):
          w16 = plsc.bitcast(eb_[e][4 + eg // 8, pl.ds(16 * (eg % 8), 16)],
                             _f32)
          base = 16 * eg
          for j in range(16):
            wj = jnp.broadcast_to(w16[j], (16,))
            for h2 in range(2):
              sl = pl.ds(16 * h2, 16)
              rows_[p][base + j, sl] = rows_[p][base + j, sl] * wj
          return __
        lax.fori_loop(0, SB // 16, _grp, None)

        for j in range(2):
          pltpu.sync_copy(rows_[p].at[pl.ds(128 * j, 128)],
                          acc.at[eb_[e].at[2 + j]], add=True)

        @pl.when(s + 2 < NSB)
        def _():
          _wait_edges(s + 2, e2)
          _fire_gathers(l, e2, p)
      return _
    lax.fori_loop(0, NSB // 4, _quad, None)
    plsc.subcore_barrier()

    cdescs = []
    for z in range(12):
      rr = sid * RPT + 256 * z
      cdescs.append(pltpu.async_copy(
          acc.at[pl.ds(rr, 256)], xs_h.at[l + 1, cid, pl.ds(rr, 256)], usem))
    cdescs.append(pltpu.async_copy(
        acc.at[pl.ds(sid * RPT + 3072, 128)],
        xs_h.at[l + 1, cid, pl.ds(sid * RPT + 3072, 128)], usem))
    for dsc in cdescs:
      dsc.wait()
    plsc.subcore_barrier()

  for ridx, idx_h in enumerate((users_h, pos_h, neg_h)):
    for t in range(SPT // 128):
      s0 = sid * SPT + 128 * t
      pltpu.sync_copy(idx_h.at[pl.ds(s0, 128)], idxb)
      if ridx > 0:
        def _off(i, _):
          v = idxb[pl.ds(16 * i, 16)]
          idxb[pl.ds(16 * i, 16)] = v + NU
          return _
        lax.fori_loop(0, 8, _off, None)
      descs = []
      for l4 in range(NL + 1):
        descs.append(pltpu.async_copy(
            xs_h.at[l4, cid].at[idxb],
            rows_[l4 // 2].at[pl.ds(128 * (l4 % 2), 128)], gsem_[l4 // 2]))
      for dsc in descs:
        dsc.wait()

      for c in range(2):
        def _mrow(r, _):
          rr = 64 * c + r
          for h2 in range(2):
            sl = pl.ds(16 * h2, 16)
            v = (rows0[rr, sl] + rows0[128 + rr, sl]
                 + rows1[rr, sl] + rows1[128 + rr, sl]) * 0.25
            outblock[r, sl] = v
          return _
        lax.fori_loop(0, 64, _mrow, None)
        pltpu.sync_copy(outblock,
                        mean_h.at[ridx, cid, pl.ds(s0 + 64 * c, 64)])

  for ridx, (idx_h, tbl_h) in enumerate(((users_h, ue_h),
                                         (pos_h, ie_h),
                                         (neg_h, ie_h))):
    s0 = cid * (B // 2) + sid * 128
    pltpu.sync_copy(idx_h.at[pl.ds(s0, 128)], idxb)
    for c in range(2):
      pltpu.async_copy(tbl_h.at[idxb.at[pl.ds(64 * c, 64)]],
                       ublock, usem).wait()
      pltpu.sync_copy(ublock, reg_h.at[ridx, pl.ds(s0 + 64 * c, 64)])


_sc_forward = pl.kernel(
    _sc_body,
    out_type=(
        jax.ShapeDtypeStruct((NL + 1, 2, NNP, H), _f32),
        jax.ShapeDtypeStruct((3, 2, B, H), _f32),
        jax.ShapeDtypeStruct((3, B, D), _f32),
    ),
    mesh=plsc.VectorSubcoreMesh(core_axis_name="c", subcore_axis_name="s",
                                num_cores=2, num_subcores=16),
    compiler_params=pltpu.CompilerParams(needs_layout_passes=False,
                                         use_tc_tiling_on_sc=False),
    scratch_types=[
        pltpu.VMEM_SHARED((NNP, H), _f32),
        pltpu.VMEM((3, D), _f32),
        pltpu.VMEM((8, D), _f32),
        pltpu.VMEM((11, D), _f32),
        pltpu.VMEM((NBLK, D), _f32),
        pltpu.VMEM((NBLK, H), _f32),
        pltpu.VMEM((SB, H), _f32),
        pltpu.VMEM((SB, H), _f32),
        pltpu.VMEM((6, 128), _i32),
        pltpu.VMEM((6, 128), _i32),
        pltpu.VMEM((6, 128), _i32),
        pltpu.VMEM((6, 128), _i32),
        pltpu.VMEM((128,), _i32),
        pltpu.SemaphoreType.DMA,
        pltpu.SemaphoreType.DMA,
        pltpu.SemaphoreType.DMA,
        pltpu.SemaphoreType.DMA,
        pltpu.SemaphoreType.DMA,
        pltpu.SemaphoreType.DMA,
        pltpu.SemaphoreType.DMA,
    ],
)


def _loss_body(mean_ref, reg_ref, out_ref):
  u = mean_ref[0]
  pi = mean_ref[1]
  ni = mean_ref[2]
  ps = jnp.sum(u * pi, axis=(0, 2))
  ns = jnp.sum(u * ni, axis=(0, 2))
  x = ps - ns
  bpr = -jnp.mean(jnp.minimum(x, 0.0) - jnp.log1p(jnp.exp(-jnp.abs(x))))
  r = reg_ref[...]
  reg = jnp.sum(r * r) / B
  out_ref[...] = jnp.reshape(bpr + DECAY * reg, (1, 1))


_tc_loss = pl.pallas_call(
    _loss_body,
    out_shape=jax.ShapeDtypeStruct((1, 1), _f32),
)


@jax.jit
def kernel(users, pos_items, neg_items, edge_index, edge_weight,
           user_gender, user_age_bucket, item_cat,
           user_emb, item_emb, gender_emb, age_emb, cat_emb):
  dst = edge_index[0].astype(_i32)
  src = edge_index[1].astype(_i32)
  pad = NEP - NE
  pad_idx = jnp.arange(pad, dtype=_i32)
  src1 = jnp.concatenate([src, pad_idx % NN])
  dst1 = jnp.concatenate([dst, NN + pad_idx % (NNP - NN)])
  w1 = jnp.concatenate([edge_weight.astype(_f32), jnp.zeros((pad,), _f32)])
  epk = jnp.concatenate(
      [src1.reshape(-1, 2, 128), dst1.reshape(-1, 2, 128),
       lax.bitcast_convert_type(w1, _i32).reshape(-1, 2, 128)], axis=1)

  _, mean_rows, reg_rows = _sc_forward(
      users.astype(_i32), pos_items.astype(_i32), neg_items.astype(_i32),
      epk,
      user_gender.astype(_i32), user_age_bucket.astype(_i32),
      item_cat.astype(_i32),
      user_emb.astype(_f32), item_emb.astype(_f32),
      gender_emb.astype(_f32), age_emb.astype(_f32), cat_emb.astype(_f32))

  loss = _tc_loss(mean_rows, reg_rows)
  return jnp.reshape(loss, ())


# speedup vs baseline: 1.2103x; 1.0191x over previous
import functools

import jax
import jax.numpy as jnp
from jax import lax
from jax.experimental import pallas as pl
from jax.experimental.pallas import tpu as pltpu
from jax.experimental.pallas import tpu_sc as plsc

NU = 25000
NI = 25000
NN = NU + NI
NNP = 51200
NE = 800000
NEP = 819200
D = 64
H = 32
B = 4096
NL = 3
DECAY = 1e-4

NBLK = 64
NUB = 391
EPT = NEP // 16
SB = 256
NSB = EPT // SB
RPT = NNP // 16
SPT = B // 16

_f32 = jnp.float32
_i32 = jnp.int32


def _sc_body(users_h, pos_h, neg_h, epk_h,
             ug_h, ua_h, ic_h, ue_h, ie_h, ge_h, ae_h, ce_h,
             xs_h, mean_h, reg_h,
             acc, gtab, atab, ctab, ublock, outblock,
             rows0, rows1, eb0, eb1, eb2, eb3, idxb,
             esem0, esem1, esem2, esem3, gsem0, gsem1, usem):
  cid = lax.axis_index("c")
  sid = lax.axis_index("s")
  cb = cid * H
  rows_ = (rows0, rows1)
  eb_ = (eb0, eb1, eb2, eb3)
  esem_ = (esem0, esem1, esem2, esem3)
  gsem_ = (gsem0, gsem1)

  pltpu.sync_copy(ge_h, gtab)
  pltpu.sync_copy(ae_h, atab)
  pltpu.sync_copy(ce_h, ctab)

  gidx_b = idxb.at[pl.ds(0, NBLK)]
  aidx_b = idxb.at[pl.ds(NBLK, NBLK)]

  def _build_user(b):
    n0 = jnp.minimum(NBLK * b, NU - NBLK)
    d1 = pltpu.async_copy(ue_h.at[pl.ds(n0, NBLK)], ublock, usem)
    d2 = pltpu.async_copy(ug_h.at[pl.ds(n0, NBLK)], gidx_b, usem)
    d3 = pltpu.async_copy(ua_h.at[pl.ds(n0, NBLK)], aidx_b, usem)
    d1.wait(); d2.wait(); d3.wait()

    def _grp(g, _):
      gv = gidx_b[pl.ds(16 * g, 16)]
      av = aidx_b[pl.ds(16 * g, 16)]
      for j in range(16):
        n = 16 * g + j
        gj = gv[j]
        aj = av[j]
        for h2 in range(2):
          sl = pl.ds(16 * h2, 16)
          outblock[n, sl] = (ublock[n, pl.ds(cb + 16 * h2, 16)]
                             + gtab[gj, pl.ds(cb + 16 * h2, 16)]
                             + atab[aj, pl.ds(cb + 16 * h2, 16)])
      return _
    lax.fori_loop(0, NBLK // 16, _grp, None)
    pltpu.sync_copy(outblock, xs_h.at[0, cid, pl.ds(n0, NBLK)])

  def _build_item(b):
    n0 = jnp.minimum(NBLK * b, NI - NBLK)
    d1 = pltpu.async_copy(ie_h.at[pl.ds(n0, NBLK)], ublock, usem)
    d2 = pltpu.async_copy(ic_h.at[pl.ds(n0, NBLK)], gidx_b, usem)
    d1.wait(); d2.wait()

    def _grp(g, _):
      cv = gidx_b[pl.ds(16 * g, 16)]
      for j in range(16):
        n = 16 * g + j
        cj = cv[j]
        for h2 in range(2):
          sl = pl.ds(16 * h2, 16)
          outblock[n, sl] = (ublock[n, pl.ds(cb + 16 * h2, 16)]
                             + ctab[cj, pl.ds(cb + 16 * h2, 16)])
      return _
    lax.fori_loop(0, NBLK // 16, _grp, None)
    pltpu.sync_copy(outblock, xs_h.at[0, cid, pl.ds(NU + n0, NBLK)])

  def _build_k(k, _):
    b = sid + 16 * k

    @pl.when(b < NUB)
    def _():
      _build_user(b)
      _build_item(b)
    return _
  lax.fori_loop(0, 25, _build_k, None)

  plsc.subcore_barrier()

  def _zb(r, _):
    outblock[r, pl.ds(0, 16)] = jnp.zeros((16,), _f32)
    outblock[r, pl.ds(16, 16)] = jnp.zeros((16,), _f32)
    return _
  lax.fori_loop(0, NBLK, _zb, None)

  def _fire_edges(s, e):
    pltpu.async_copy(epk_h.at[sid * NSB + s], eb_[e], esem_[e])

  def _wait_edges(s, e):
    pltpu.make_async_copy(epk_h.at[sid * NSB + s], eb_[e], esem_[e]).wait()

  def _fire_gathers(l, e, p):
    for j in range(2):
      pltpu.async_copy(xs_h.at[l, cid].at[eb_[e].at[j]],
                       rows_[p].at[pl.ds(128 * j, 128)], gsem_[p])

  def _wait_gathers(l, e, p):
    for j in range(2):
      pltpu.make_async_copy(xs_h.at[l, cid].at[eb_[e].at[j]],
                            rows_[p].at[pl.ds(128 * j, 128)], gsem_[p]).wait()

  for l in range(NL):
    zdescs = []
    for z in range(RPT // NBLK):
      zdescs.append(pltpu.async_copy(
          outblock, acc.at[pl.ds(sid * RPT + NBLK * z, NBLK)], usem))
    for dsc in zdescs:
      dsc.wait()
    plsc.subcore_barrier()

    _fire_edges(0, 0)
    _fire_edges(1, 1)
    _wait_edges(0, 0)
    _fire_gathers(l, 0, 0)
    _wait_edges(1, 1)
    _fire_gathers(l, 1, 1)

    def _quad(i, _):
      for u in range(4):
        s = 4 * i + u
        p = u % 2
        e = u
        e2 = (u + 2) % 4

        @pl.when(s + 2 < NSB)
        def _():
          _fire_edges(s + 2, e2)

        _wait_gathers(l, e, p)

        def _grp(eg, __):
          w16 = plsc.bitcast(eb_[e][4 + eg // 8, pl.ds(16 * (eg % 8), 16)],
                             _f32)
          base = 16 * eg
          for j in range(16):
            wj = jnp.broadcast_to(w16[j], (16,))
            for h2 in range(2):
              sl = pl.ds(16 * h2, 16)
              rows_[p][base + j, sl] = rows_[p][base + j, sl] * wj
          return __
        lax.fori_loop(0, SB // 16, _grp, None)

        xd = []
        for j in range(2):
          xd.append(pltpu.async_copy(rows_[p].at[pl.ds(128 * j, 128)],
                                     acc.at[eb_[e].at[2 + j]], usem,
                                     add=True))
        for dsc in xd:
          dsc.wait()

        @pl.when(s + 2 < NSB)
        def _():
          _wait_edges(s + 2, e2)
          _fire_gathers(l, e2, p)
      return _
    lax.fori_loop(0, NSB // 4, _quad, None)
    plsc.subcore_barrier()

    cdescs = []
    for z in range(12):
      rr = sid * RPT + 256 * z
      cdescs.append(pltpu.async_copy(
          acc.at[pl.ds(rr, 256)], xs_h.at[l + 1, cid, pl.ds(rr, 256)], usem))
    cdescs.append(pltpu.async_copy(
        acc.at[pl.ds(sid * RPT + 3072, 128)],
        xs_h.at[l + 1, cid, pl.ds(sid * RPT + 3072, 128)], usem))
    for dsc in cdescs:
      dsc.wait()
    plsc.subcore_barrier()

  for ridx, idx_h in enumerate((users_h, pos_h, neg_h)):
    for t in range(SPT // 128):
      s0 = sid * SPT + 128 * t
      pltpu.sync_copy(idx_h.at[pl.ds(s0, 128)], idxb)
      if ridx > 0:
        def _off(i, _):
          v = idxb[pl.ds(16 * i, 16)]
          idxb[pl.ds(16 * i, 16)] = v + NU
          return _
        lax.fori_loop(0, 8, _off, None)
      descs = []
      for l4 in range(NL + 1):
        descs.append(pltpu.async_copy(
            xs_h.at[l4, cid].at[idxb],
            rows_[l4 // 2].at[pl.ds(128 * (l4 % 2), 128)], gsem_[l4 // 2]))
      for dsc in descs:
        dsc.wait()

      for c in range(2):
        def _mrow(r, _):
          rr = 64 * c + r
          for h2 in range(2):
            sl = pl.ds(16 * h2, 16)
            v = (rows0[rr, sl] + rows0[128 + rr, sl]
                 + rows1[rr, sl] + rows1[128 + rr, sl]) * 0.25
            outblock[r, sl] = v
          return _
        lax.fori_loop(0, 64, _mrow, None)
        pltpu.sync_copy(outblock,
                        mean_h.at[ridx, cid, pl.ds(s0 + 64 * c, 64)])

  for ridx, (idx_h, tbl_h) in enumerate(((users_h, ue_h),
                                         (pos_h, ie_h),
                                         (neg_h, ie_h))):
    s0 = cid * (B // 2) + sid * 128
    pltpu.sync_copy(idx_h.at[pl.ds(s0, 128)], idxb)
    for c in range(2):
      pltpu.async_copy(tbl_h.at[idxb.at[pl.ds(64 * c, 64)]],
                       ublock, usem).wait()
      pltpu.sync_copy(ublock, reg_h.at[ridx, pl.ds(s0 + 64 * c, 64)])


_sc_forward = pl.kernel(
    _sc_body,
    out_type=(
        jax.ShapeDtypeStruct((NL + 1, 2, NNP, H), _f32),
        jax.ShapeDtypeStruct((3, 2, B, H), _f32),
        jax.ShapeDtypeStruct((3, B, D), _f32),
    ),
    mesh=plsc.VectorSubcoreMesh(core_axis_name="c", subcore_axis_name="s",
                                num_cores=2, num_subcores=16),
    compiler_params=pltpu.CompilerParams(needs_layout_passes=False,
                                         use_tc_tiling_on_sc=False),
    scratch_types=[
        pltpu.VMEM_SHARED((NNP, H), _f32),
        pltpu.VMEM((3, D), _f32),
        pltpu.VMEM((8, D), _f32),
        pltpu.VMEM((11, D), _f32),
        pltpu.VMEM((NBLK, D), _f32),
        pltpu.VMEM((NBLK, H), _f32),
        pltpu.VMEM((SB, H), _f32),
        pltpu.VMEM((SB, H), _f32),
        pltpu.VMEM((6, 128), _i32),
        pltpu.VMEM((6, 128), _i32),
        pltpu.VMEM((6, 128), _i32),
        pltpu.VMEM((6, 128), _i32),
        pltpu.VMEM((128,), _i32),
        pltpu.SemaphoreType.DMA,
        pltpu.SemaphoreType.DMA,
        pltpu.SemaphoreType.DMA,
        pltpu.SemaphoreType.DMA,
        pltpu.SemaphoreType.DMA,
        pltpu.SemaphoreType.DMA,
        pltpu.SemaphoreType.DMA,
    ],
)


def _loss_body(mean_ref, reg_ref, out_ref):
  u = mean_ref[0]
  pi = mean_ref[1]
  ni = mean_ref[2]
  ps = jnp.sum(u * pi, axis=(0, 2))
  ns = jnp.sum(u * ni, axis=(0, 2))
  x = ps - ns
  bpr = -jnp.mean(jnp.minimum(x, 0.0) - jnp.log1p(jnp.exp(-jnp.abs(x))))
  r = reg_ref[...]
  reg = jnp.sum(r * r) / B
  out_ref[...] = jnp.reshape(bpr + DECAY * reg, (1, 1))


_tc_loss = pl.pallas_call(
    _loss_body,
    out_shape=jax.ShapeDtypeStruct((1, 1), _f32),
)


@jax.jit
def kernel(users, pos_items, neg_items, edge_index, edge_weight,
           user_gender, user_age_bucket, item_cat,
           user_emb, item_emb, gender_emb, age_emb, cat_emb):
  dst = edge_index[0].astype(_i32)
  src = edge_index[1].astype(_i32)
  pad = NEP - NE
  pad_idx = jnp.arange(pad, dtype=_i32)
  src1 = jnp.concatenate([src, pad_idx % NN])
  dst1 = jnp.concatenate([dst, NN + pad_idx % (NNP - NN)])
  w1 = jnp.concatenate([edge_weight.astype(_f32), jnp.zeros((pad,), _f32)])
  epk = jnp.concatenate(
      [src1.reshape(-1, 2, 128), dst1.reshape(-1, 2, 128),
       lax.bitcast_convert_type(w1, _i32).reshape(-1, 2, 128)], axis=1)

  _, mean_rows, reg_rows = _sc_forward(
      users.astype(_i32), pos_items.astype(_i32), neg_items.astype(_i32),
      epk,
      user_gender.astype(_i32), user_age_bucket.astype(_i32),
      item_cat.astype(_i32),
      user_emb.astype(_f32), item_emb.astype(_f32),
      gender_emb.astype(_f32), age_emb.astype(_f32), cat_emb.astype(_f32))

  loss = _tc_loss(mean_rows, reg_rows)
  return jnp.reshape(loss, ())
